# baseline (speedup 1.0000x reference)
import functools

import jax
import jax.numpy as jnp
from jax import lax
from jax.experimental import pallas as pl
from jax.experimental.pallas import tpu as pltpu
from jax.experimental.pallas import tpu_sc as plsc

NN = 10000
NE = 320000
FD = 128
HD = 256
CD = 40
DP = 48
NP = 10240
BR = 512
GR = NP // BR

NC = 2
NS = 16
NW = NC * NS
CHW = 128
NCH = 79
EPT = NCH * CHW
EPAD = NW * EPT
RPT = NP // NS
ZR = 128

_mesh = plsc.VectorSubcoreMesh(core_axis_name="c", subcore_axis_name="s")
_sc_params = pltpu.CompilerParams(
    needs_layout_passes=False, use_tc_tiling_on_sc=False
)


@functools.partial(
    pl.kernel,
    mesh=_mesh,
    out_type=jax.ShapeDtypeStruct((NW, NP), jnp.float32),
    scratch_types=[
        pltpu.VMEM((EPT,), jnp.int32),
        pltpu.VMEM((NP,), jnp.float32),
    ],
    compiler_params=_sc_params,
)
def _deg_kernel(dst_hbm, degp_hbm, dbuf, degloc):
    c = lax.axis_index("c")
    s = lax.axis_index("s")
    wid = c * NS + s
    zero16 = jnp.zeros((16,), jnp.float32)
    ones16 = jnp.ones((16,), jnp.float32)

    def zbody(i, carry):
        degloc[pl.ds(i * 16, 16)] = zero16
        return carry

    lax.fori_loop(0, NP // 16, zbody, 0)

    pltpu.sync_copy(dst_hbm.at[pl.ds(wid * EPT, EPT)], dbuf)

    def inner(i, icarry):
        idx = dbuf[pl.ds(i * 16, 16)]
        plsc.addupdate_scatter(degloc, [idx], ones16)
        return icarry

    lax.fori_loop(0, EPT // 16, inner, 0)
    pltpu.sync_copy(degloc, degp_hbm.at[wid])


@functools.partial(
    pl.kernel,
    mesh=_mesh,
    out_type=jax.ShapeDtypeStruct((NC * NP, DP), jnp.float32),
    scratch_types=[
        pltpu.VMEM((NCH, CHW), jnp.int32),
        pltpu.VMEM((NCH, CHW), jnp.int32),
        [pltpu.VMEM((CHW, DP), jnp.float32)] * 4,
        pltpu.VMEM((ZR, DP), jnp.float32),
        pltpu.VMEM_SHARED((NP, DP), jnp.float32),
        [pltpu.SemaphoreType.DMA] * 4,
    ],
    compiler_params=_sc_params,
)
def _hop_kernel(h_hbm, src_hbm, dst_hbm, out_hbm, srcb, dstb, rows,
                zbuf, accum, gsem):
    c = lax.axis_index("c")
    s = lax.axis_index("s")
    wid = c * NS + s
    zero16 = jnp.zeros((16,), jnp.float32)

    def zrow(i, carry):
        for jcol in range(DP // 16):
            zbuf[i, pl.ds(jcol * 16, 16)] = zero16
        return carry

    lax.fori_loop(0, ZR, zrow, 0)

    rbase = s * RPT
    for i in range(RPT // ZR):
        pltpu.sync_copy(zbuf, accum.at[pl.ds(rbase + i * ZR, ZR)])

    pltpu.sync_copy(src_hbm.at[wid], srcb)
    pltpu.sync_copy(dst_hbm.at[wid], dstb)
    plsc.subcore_barrier()

    pltpu.async_copy(h_hbm.at[srcb.at[0]], rows[0], gsem[0])

    def pair(i, carry):
        j = i * 2
        pltpu.async_copy(h_hbm.at[srcb.at[j + 1]], rows[1], gsem[1])
        pltpu.make_async_copy(h_hbm.at[srcb.at[j]], rows[0], gsem[0]).wait()
        pltpu.sync_copy(rows[0], accum.at[dstb.at[j]], add=True)
        pltpu.async_copy(h_hbm.at[srcb.at[j + 2]], rows[0], gsem[0])
        pltpu.make_async_copy(h_hbm.at[srcb.at[j + 1]], rows[1], gsem[1]).wait()
        pltpu.sync_copy(rows[1], accum.at[dstb.at[j + 1]], add=True)
        return carry

    lax.fori_loop(0, NCH // 2, pair, 0)
    pltpu.make_async_copy(h_hbm.at[srcb.at[NCH - 1]], rows[0], gsem[0]).wait()
    pltpu.sync_copy(rows[0], accum.at[dstb.at[NCH - 1]], add=True)
    plsc.subcore_barrier()
    pltpu.sync_copy(
        accum.at[pl.ds(rbase, RPT)],
        out_hbm.at[pl.ds(c * NP + rbase, RPT)],
    )


def _tc0_body(x_ref, wsgc_ref, wmlp_ref, z_ref):
    wf = jnp.dot(wsgc_ref[...], wmlp_ref[...], preferred_element_type=jnp.float32)
    z_ref[...] = jnp.dot(x_ref[...], wf, preferred_element_type=jnp.float32)


def _tc1_body(z_ref, degp_ref, h0_ref):
    deg = jnp.sum(degp_ref[...], axis=0) + 1.0
    dinv = lax.rsqrt(deg)
    h0_ref[...] = z_ref[...] * dinv[:, None]


def _tc2_body(sa_ref, sb_ref, h0_ref, degp_ref, h1_ref):
    deg = jnp.sum(degp_ref[...], axis=0) + 1.0
    ssum = sa_ref[...] + sb_ref[...] + h0_ref[...]
    h1_ref[...] = ssum / deg[:, None]


def _tc3_body(sa_ref, sb_ref, h1_ref, degp_ref, wmlp_ref, bsgc_ref, bmlp_ref, out_ref):
    deg = jnp.sum(degp_ref[...], axis=0) + 1.0
    dinv = lax.rsqrt(deg)
    r = (sa_ref[...] + sb_ref[...] + h1_ref[...]) * dinv[:, None]
    bf = jnp.dot(bsgc_ref[...], wmlp_ref[...], preferred_element_type=jnp.float32) + bmlp_ref[...]
    out_ref[...] = r[:, :CD] + bf


_row_spec = pl.BlockSpec((BR, DP), lambda i: (i, 0))
_dega_spec = pl.BlockSpec((NW, BR), lambda i: (0, i))
_sa_spec = pl.BlockSpec((BR, DP), lambda i: (i, 0))
_sb_spec = pl.BlockSpec((BR, DP), lambda i: (i + GR, 0))

_tc0 = pl.pallas_call(
    _tc0_body,
    grid=(GR,),
    in_specs=[
        pl.BlockSpec((BR, FD), lambda i: (i, 0)),
        pl.BlockSpec((FD, HD), lambda i: (0, 0)),
        pl.BlockSpec((HD, DP), lambda i: (0, 0)),
    ],
    out_specs=_row_spec,
    out_shape=jax.ShapeDtypeStruct((NN, DP), jnp.float32),
)

_tc1 = pl.pallas_call(
    _tc1_body,
    grid=(GR,),
    in_specs=[_row_spec, _dega_spec],
    out_specs=_row_spec,
    out_shape=jax.ShapeDtypeStruct((NN, DP), jnp.float32),
)

_tc2 = pl.pallas_call(
    _tc2_body,
    grid=(GR,),
    in_specs=[_sa_spec, _sb_spec, _row_spec, _dega_spec],
    out_specs=_row_spec,
    out_shape=jax.ShapeDtypeStruct((NN, DP), jnp.float32),
)

_tc3 = pl.pallas_call(
    _tc3_body,
    grid=(GR,),
    in_specs=[
        _sa_spec,
        _sb_spec,
        _row_spec,
        _dega_spec,
        pl.BlockSpec((HD, CD), lambda i: (0, 0)),
        pl.BlockSpec((1, HD), lambda i: (0, 0)),
        pl.BlockSpec((1, CD), lambda i: (0, 0)),
    ],
    out_specs=pl.BlockSpec((BR, CD), lambda i: (i, 0)),
    out_shape=jax.ShapeDtypeStruct((NN, CD), jnp.float32),
)


def kernel(x, edge_index, W_sgc, b_sgc, W_mlp, b_mlp):
    src = edge_index[0]
    dst = edge_index[1]
    wmlp_pad = jnp.pad(W_mlp, ((0, 0), (0, DP - CD)))
    npad = EPAD - NE
    pads = jnp.arange(npad, dtype=jnp.int32)
    srcp = jnp.concatenate([src, pads % NN])
    dstp = jnp.concatenate([dst, NN + pads % (NP - NN)])
    src3 = srcp.reshape(NW, NCH, CHW)
    dst3 = dstp.reshape(NW, NCH, CHW)

    z = _tc0(x, W_sgc, wmlp_pad)
    degp = _deg_kernel(dstp)
    h0 = _tc1(z, degp)
    s1 = _hop_kernel(h0, src3, dst3)
    h1 = _tc2(s1, s1, h0, degp)
    s2 = _hop_kernel(h1, src3, dst3)
    return _tc3(s2, s2, h1, degp, W_mlp,
                b_sgc.reshape(1, HD), b_mlp.reshape(1, CD))


# speedup vs baseline: 1.1449x; 1.1449x over previous
import functools

import jax
import jax.numpy as jnp
from jax import lax
from jax.experimental import pallas as pl
from jax.experimental.pallas import tpu as pltpu
from jax.experimental.pallas import tpu_sc as plsc

NN = 10000
NE = 320000
FD = 128
HD = 256
CD = 40
DP = 48
NP = 10240
BR = 512
GR = NP // BR

NC = 2
NS = 16
NW = NC * NS
CHW = 512
NCH = 20
EPT = NCH * CHW
EPAD = NW * EPT
RPT = NP // NS
ZR = 128

_mesh = plsc.VectorSubcoreMesh(core_axis_name="c", subcore_axis_name="s")
_sc_params = pltpu.CompilerParams(
    needs_layout_passes=False, use_tc_tiling_on_sc=False
)


@functools.partial(
    pl.kernel,
    mesh=_mesh,
    out_type=jax.ShapeDtypeStruct((NW, NP), jnp.float32),
    scratch_types=[
        pltpu.VMEM((EPT,), jnp.int32),
        pltpu.VMEM((NP,), jnp.float32),
    ],
    compiler_params=_sc_params,
)
def _deg_kernel(dst_hbm, degp_hbm, dbuf, degloc):
    c = lax.axis_index("c")
    s = lax.axis_index("s")
    wid = c * NS + s
    zero16 = jnp.zeros((16,), jnp.float32)
    ones16 = jnp.ones((16,), jnp.float32)

    def zbody(i, carry):
        degloc[pl.ds(i * 16, 16)] = zero16
        return carry

    lax.fori_loop(0, NP // 16, zbody, 0)

    pltpu.sync_copy(dst_hbm.at[pl.ds(wid * EPT, EPT)], dbuf)

    def inner(i, icarry):
        idx = dbuf[pl.ds(i * 16, 16)]
        plsc.addupdate_scatter(degloc, [idx], ones16)
        return icarry

    lax.fori_loop(0, EPT // 16, inner, 0)
    pltpu.sync_copy(degloc, degp_hbm.at[wid])


@functools.partial(
    pl.kernel,
    mesh=_mesh,
    out_type=jax.ShapeDtypeStruct((NC * NP, DP), jnp.float32),
    scratch_types=[
        pltpu.VMEM((NCH, CHW), jnp.int32),
        pltpu.VMEM((NCH, CHW), jnp.int32),
        [pltpu.VMEM((CHW, DP), jnp.float32)] * 2,
        pltpu.VMEM((ZR, DP), jnp.float32),
        pltpu.VMEM_SHARED((NP, DP), jnp.float32),
        [pltpu.SemaphoreType.DMA] * 2,
    ],
    compiler_params=_sc_params,
)
def _hop_kernel(h_hbm, src_hbm, dst_hbm, out_hbm, srcb, dstb, rows,
                zbuf, accum, gsem):
    c = lax.axis_index("c")
    s = lax.axis_index("s")
    wid = c * NS + s
    zero16 = jnp.zeros((16,), jnp.float32)

    def zrow(i, carry):
        for jcol in range(DP // 16):
            zbuf[i, pl.ds(jcol * 16, 16)] = zero16
        return carry

    lax.fori_loop(0, ZR, zrow, 0)

    rbase = s * RPT
    for i in range(RPT // ZR):
        pltpu.sync_copy(zbuf, accum.at[pl.ds(rbase + i * ZR, ZR)])

    pltpu.sync_copy(src_hbm.at[wid], srcb)
    pltpu.sync_copy(dst_hbm.at[wid], dstb)
    plsc.subcore_barrier()

    pltpu.async_copy(h_hbm.at[srcb.at[0]], rows[0], gsem[0])

    def pair(i, carry):
        j = i * 2
        pltpu.async_copy(h_hbm.at[srcb.at[j + 1]], rows[1], gsem[1])
        pltpu.make_async_copy(h_hbm.at[srcb.at[j]], rows[0], gsem[0]).wait()
        pltpu.sync_copy(rows[0], accum.at[dstb.at[j]], add=True)

        @pl.when(j + 2 < NCH)
        def _():
            pltpu.async_copy(h_hbm.at[srcb.at[j + 2]], rows[0], gsem[0])

        pltpu.make_async_copy(h_hbm.at[srcb.at[j + 1]], rows[1], gsem[1]).wait()
        pltpu.sync_copy(rows[1], accum.at[dstb.at[j + 1]], add=True)
        return carry

    lax.fori_loop(0, NCH // 2, pair, 0)
    plsc.subcore_barrier()
    pltpu.sync_copy(
        accum.at[pl.ds(rbase, RPT)],
        out_hbm.at[pl.ds(c * NP + rbase, RPT)],
    )


def _tc0_body(x_ref, wsgc_ref, wmlp_ref, z_ref):
    wf = jnp.dot(wsgc_ref[...], wmlp_ref[...], preferred_element_type=jnp.float32)
    z_ref[...] = jnp.dot(x_ref[...], wf, preferred_element_type=jnp.float32)


def _tc1_body(z_ref, degp_ref, h0_ref):
    deg = jnp.sum(degp_ref[...], axis=0) + 1.0
    dinv = lax.rsqrt(deg)
    h0_ref[...] = z_ref[...] * dinv[:, None]


def _tc2_body(sa_ref, sb_ref, h0_ref, degp_ref, h1_ref):
    deg = jnp.sum(degp_ref[...], axis=0) + 1.0
    ssum = sa_ref[...] + sb_ref[...] + h0_ref[...]
    h1_ref[...] = ssum / deg[:, None]


def _tc3_body(sa_ref, sb_ref, h1_ref, degp_ref, wmlp_ref, bsgc_ref, bmlp_ref, out_ref):
    deg = jnp.sum(degp_ref[...], axis=0) + 1.0
    dinv = lax.rsqrt(deg)
    r = (sa_ref[...] + sb_ref[...] + h1_ref[...]) * dinv[:, None]
    bf = jnp.dot(bsgc_ref[...], wmlp_ref[...], preferred_element_type=jnp.float32) + bmlp_ref[...]
    out_ref[...] = r[:, :CD] + bf


_row_spec = pl.BlockSpec((BR, DP), lambda i: (i, 0))
_dega_spec = pl.BlockSpec((NW, BR), lambda i: (0, i))
_sa_spec = pl.BlockSpec((BR, DP), lambda i: (i, 0))
_sb_spec = pl.BlockSpec((BR, DP), lambda i: (i + GR, 0))

_tc0 = pl.pallas_call(
    _tc0_body,
    grid=(GR,),
    in_specs=[
        pl.BlockSpec((BR, FD), lambda i: (i, 0)),
        pl.BlockSpec((FD, HD), lambda i: (0, 0)),
        pl.BlockSpec((HD, DP), lambda i: (0, 0)),
    ],
    out_specs=_row_spec,
    out_shape=jax.ShapeDtypeStruct((NN, DP), jnp.float32),
)

_tc1 = pl.pallas_call(
    _tc1_body,
    grid=(GR,),
    in_specs=[_row_spec, _dega_spec],
    out_specs=_row_spec,
    out_shape=jax.ShapeDtypeStruct((NN, DP), jnp.float32),
)

_tc2 = pl.pallas_call(
    _tc2_body,
    grid=(GR,),
    in_specs=[_sa_spec, _sb_spec, _row_spec, _dega_spec],
    out_specs=_row_spec,
    out_shape=jax.ShapeDtypeStruct((NN, DP), jnp.float32),
)

_tc3 = pl.pallas_call(
    _tc3_body,
    grid=(GR,),
    in_specs=[
        _sa_spec,
        _sb_spec,
        _row_spec,
        _dega_spec,
        pl.BlockSpec((HD, CD), lambda i: (0, 0)),
        pl.BlockSpec((1, HD), lambda i: (0, 0)),
        pl.BlockSpec((1, CD), lambda i: (0, 0)),
    ],
    out_specs=pl.BlockSpec((BR, CD), lambda i: (i, 0)),
    out_shape=jax.ShapeDtypeStruct((NN, CD), jnp.float32),
)


def kernel(x, edge_index, W_sgc, b_sgc, W_mlp, b_mlp):
    src = edge_index[0]
    dst = edge_index[1]
    wmlp_pad = jnp.pad(W_mlp, ((0, 0), (0, DP - CD)))
    npad = EPAD - NE
    pads = jnp.arange(npad, dtype=jnp.int32)
    srcp = jnp.concatenate([src, pads % NN])
    dstp = jnp.concatenate([dst, NN + pads % (NP - NN)])
    src3 = srcp.reshape(NW, NCH, CHW)
    dst3 = dstp.reshape(NW, NCH, CHW)

    z = _tc0(x, W_sgc, wmlp_pad)
    degp = _deg_kernel(dstp)
    h0 = _tc1(z, degp)
    s1 = _hop_kernel(h0, src3, dst3)
    h1 = _tc2(s1, s1, h0, degp)
    s2 = _hop_kernel(h1, src3, dst3)
    return _tc3(s2, s2, h1, degp, W_mlp,
                b_sgc.reshape(1, HD), b_mlp.reshape(1, CD))


# speedup vs baseline: 1.1736x; 1.0251x over previous
import functools

import jax
import jax.numpy as jnp
from jax import lax
from jax.experimental import pallas as pl
from jax.experimental.pallas import tpu as pltpu
from jax.experimental.pallas import tpu_sc as plsc

NN = 10000
NE = 320000
FD = 128
HD = 256
CD = 40
DP = 48
NP = 10240
BR = 512
GR = NP // BR

NC = 2
NS = 16
NW = NC * NS
CHW = 512
NCH = 20
EPT = NCH * CHW
EPAD = NW * EPT
RPT = NP // NS
ZR = 128

_mesh = plsc.VectorSubcoreMesh(core_axis_name="c", subcore_axis_name="s")
_sc_params = pltpu.CompilerParams(
    needs_layout_passes=False, use_tc_tiling_on_sc=False
)


@functools.partial(
    pl.kernel,
    mesh=_mesh,
    out_type=jax.ShapeDtypeStruct((NW, NP), jnp.float32),
    scratch_types=[
        pltpu.VMEM((EPT,), jnp.int32),
        pltpu.VMEM((NP,), jnp.float32),
    ],
    compiler_params=_sc_params,
)
def _deg_kernel(dst_hbm, degp_hbm, dbuf, degloc):
    c = lax.axis_index("c")
    s = lax.axis_index("s")
    wid = c * NS + s
    zero16 = jnp.zeros((16,), jnp.float32)
    ones16 = jnp.ones((16,), jnp.float32)

    def zbody(i, carry):
        degloc[pl.ds(i * 16, 16)] = zero16
        return carry

    lax.fori_loop(0, NP // 16, zbody, 0)

    pltpu.sync_copy(dst_hbm.at[pl.ds(wid * EPT, EPT)], dbuf)

    def inner(i, icarry):
        idx = dbuf[pl.ds(i * 16, 16)]
        plsc.addupdate_scatter(degloc, [idx], ones16)
        return icarry

    lax.fori_loop(0, EPT // 16, inner, 0)
    pltpu.sync_copy(degloc, degp_hbm.at[wid])


@functools.partial(
    pl.kernel,
    mesh=_mesh,
    out_type=jax.ShapeDtypeStruct((NC * NP, DP), jnp.float32),
    scratch_types=[
        pltpu.VMEM((NCH, CHW), jnp.int32),
        pltpu.VMEM((NCH, CHW), jnp.int32),
        [pltpu.VMEM((CHW, DP), jnp.float32)] * 2,
        pltpu.VMEM((ZR, DP), jnp.float32),
        pltpu.VMEM_SHARED((NP, DP), jnp.float32),
        [pltpu.SemaphoreType.DMA] * 2,
    ],
    compiler_params=_sc_params,
)
def _hop_kernel(h_hbm, src_hbm, dst_hbm, out_hbm, srcb, dstb, rows,
                zbuf, accum, gsem):
    c = lax.axis_index("c")
    s = lax.axis_index("s")
    wid = c * NS + s
    zero16 = jnp.zeros((16,), jnp.float32)

    def zrow(i, carry):
        for jcol in range(DP // 16):
            zbuf[i, pl.ds(jcol * 16, 16)] = zero16
        return carry

    lax.fori_loop(0, ZR, zrow, 0)

    rbase = s * RPT
    for i in range(RPT // ZR):
        pltpu.sync_copy(zbuf, accum.at[pl.ds(rbase + i * ZR, ZR)])

    pltpu.sync_copy(src_hbm.at[wid], srcb)
    pltpu.sync_copy(dst_hbm.at[wid], dstb)
    plsc.subcore_barrier()

    pltpu.async_copy(h_hbm.at[srcb.at[0]], rows[0], gsem[0])

    def pair(i, carry):
        j = i * 2
        pltpu.async_copy(h_hbm.at[srcb.at[j + 1]], rows[1], gsem[1])
        pltpu.make_async_copy(h_hbm.at[srcb.at[j]], rows[0], gsem[0]).wait()
        pltpu.sync_copy(rows[0], accum.at[dstb.at[j]], add=True)

        @pl.when(j + 2 < NCH)
        def _():
            pltpu.async_copy(h_hbm.at[srcb.at[j + 2]], rows[0], gsem[0])

        pltpu.make_async_copy(h_hbm.at[srcb.at[j + 1]], rows[1], gsem[1]).wait()
        pltpu.sync_copy(rows[1], accum.at[dstb.at[j + 1]], add=True)
        return carry

    lax.fori_loop(0, NCH // 2, pair, 0)
    plsc.subcore_barrier()
    pltpu.sync_copy(
        accum.at[pl.ds(rbase, RPT)],
        out_hbm.at[pl.ds(c * NP + rbase, RPT)],
    )


def _tc1_body(x_ref, wsgc_ref, wmlp_ref, degp_ref, h0_ref):
    deg = jnp.sum(degp_ref[...], axis=0) + 1.0
    dinv = lax.rsqrt(deg)
    wf = jnp.dot(wsgc_ref[...], wmlp_ref[...], preferred_element_type=jnp.float32)
    z = jnp.dot(x_ref[...], wf, preferred_element_type=jnp.float32)
    h0_ref[...] = z * dinv[:, None]


def _tc2_body(sa_ref, sb_ref, h0_ref, degp_ref, h1_ref):
    deg = jnp.sum(degp_ref[...], axis=0) + 1.0
    ssum = sa_ref[...] + sb_ref[...] + h0_ref[...]
    h1_ref[...] = ssum / deg[:, None]


def _tc3_body(sa_ref, sb_ref, h1_ref, degp_ref, wmlp_ref, bsgc_ref, bmlp_ref, out_ref):
    deg = jnp.sum(degp_ref[...], axis=0) + 1.0
    dinv = lax.rsqrt(deg)
    r = (sa_ref[...] + sb_ref[...] + h1_ref[...]) * dinv[:, None]
    bf = jnp.dot(bsgc_ref[...], wmlp_ref[...], preferred_element_type=jnp.float32) + bmlp_ref[...]
    out_ref[...] = r[:, :CD] + bf


_row_spec = pl.BlockSpec((BR, DP), lambda i: (i, 0))
_dega_spec = pl.BlockSpec((NW, BR), lambda i: (0, i))
_sa_spec = pl.BlockSpec((BR, DP), lambda i: (i, 0))
_sb_spec = pl.BlockSpec((BR, DP), lambda i: (i + GR, 0))

_tc1 = pl.pallas_call(
    _tc1_body,
    grid=(GR,),
    in_specs=[
        pl.BlockSpec((BR, FD), lambda i: (i, 0)),
        pl.BlockSpec((FD, HD), lambda i: (0, 0)),
        pl.BlockSpec((HD, DP), lambda i: (0, 0)),
        _dega_spec,
    ],
    out_specs=_row_spec,
    out_shape=jax.ShapeDtypeStruct((NN, DP), jnp.float32),
)

_tc2 = pl.pallas_call(
    _tc2_body,
    grid=(GR,),
    in_specs=[_sa_spec, _sb_spec, _row_spec, _dega_spec],
    out_specs=_row_spec,
    out_shape=jax.ShapeDtypeStruct((NN, DP), jnp.float32),
)

_tc3 = pl.pallas_call(
    _tc3_body,
    grid=(GR,),
    in_specs=[
        _sa_spec,
        _sb_spec,
        _row_spec,
        _dega_spec,
        pl.BlockSpec((HD, CD), lambda i: (0, 0)),
        pl.BlockSpec((1, HD), lambda i: (0, 0)),
        pl.BlockSpec((1, CD), lambda i: (0, 0)),
    ],
    out_specs=pl.BlockSpec((BR, CD), lambda i: (i, 0)),
    out_shape=jax.ShapeDtypeStruct((NN, CD), jnp.float32),
)


def kernel(x, edge_index, W_sgc, b_sgc, W_mlp, b_mlp):
    src = edge_index[0]
    dst = edge_index[1]
    wmlp_pad = jnp.pad(W_mlp, ((0, 0), (0, DP - CD)))
    npad = EPAD - NE
    pads = jnp.arange(npad, dtype=jnp.int32)
    srcp = jnp.concatenate([src, pads % NN])
    dstp = jnp.concatenate([dst, NN + pads % (NP - NN)])
    src3 = srcp.reshape(NW, NCH, CHW)
    dst3 = dstp.reshape(NW, NCH, CHW)

    degp = _deg_kernel(dstp)
    h0 = _tc1(x, W_sgc, wmlp_pad, degp)
    s1 = _hop_kernel(h0, src3, dst3)
    h1 = _tc2(s1, s1, h0, degp)
    s2 = _hop_kernel(h1, src3, dst3)
    return _tc3(s2, s2, h1, degp, W_mlp,
                b_sgc.reshape(1, HD), b_mlp.reshape(1, CD))


# speedup vs baseline: 1.1764x; 1.0024x over previous
import functools

import jax
import jax.numpy as jnp
from jax import lax
from jax.experimental import pallas as pl
from jax.experimental.pallas import tpu as pltpu
from jax.experimental.pallas import tpu_sc as plsc

NN = 10000
NE = 320000
FD = 128
HD = 256
CD = 40
DP = 48
NP = 10240
BR = 512
GR = NP // BR

NC = 2
NS = 16
NW = NC * NS
CHW = 640
NCH = 16
EPT = NCH * CHW
EPAD = NW * EPT
RPT = NP // NS
ZR = 128

_mesh = plsc.VectorSubcoreMesh(core_axis_name="c", subcore_axis_name="s")
_sc_params = pltpu.CompilerParams(
    needs_layout_passes=False, use_tc_tiling_on_sc=False
)


@functools.partial(
    pl.kernel,
    mesh=_mesh,
    out_type=jax.ShapeDtypeStruct((NW, NP), jnp.float32),
    scratch_types=[
        pltpu.VMEM((EPT,), jnp.int32),
        pltpu.VMEM((NP,), jnp.float32),
    ],
    compiler_params=_sc_params,
)
def _deg_kernel(dst_hbm, degp_hbm, dbuf, degloc):
    c = lax.axis_index("c")
    s = lax.axis_index("s")
    wid = c * NS + s
    zero16 = jnp.zeros((16,), jnp.float32)
    ones16 = jnp.ones((16,), jnp.float32)

    def zbody(i, carry):
        degloc[pl.ds(i * 16, 16)] = zero16
        return carry

    lax.fori_loop(0, NP // 16, zbody, 0)

    pltpu.sync_copy(dst_hbm.at[pl.ds(wid * EPT, EPT)], dbuf)

    def inner(i, icarry):
        idx = dbuf[pl.ds(i * 16, 16)]
        plsc.addupdate_scatter(degloc, [idx], ones16)
        return icarry

    lax.fori_loop(0, EPT // 16, inner, 0)
    pltpu.sync_copy(degloc, degp_hbm.at[wid])


@functools.partial(
    pl.kernel,
    mesh=_mesh,
    out_type=jax.ShapeDtypeStruct((NC * NP, DP), jnp.float32),
    scratch_types=[
        pltpu.VMEM((NCH, CHW), jnp.int32),
        pltpu.VMEM((NCH, CHW), jnp.int32),
        [pltpu.VMEM((CHW, DP), jnp.float32)] * 2,
        pltpu.VMEM((ZR, DP), jnp.float32),
        pltpu.VMEM_SHARED((NP, DP), jnp.float32),
        [pltpu.SemaphoreType.DMA] * 2,
    ],
    compiler_params=_sc_params,
)
def _hop_kernel(h_hbm, src_hbm, dst_hbm, out_hbm, srcb, dstb, rows,
                zbuf, accum, gsem):
    c = lax.axis_index("c")
    s = lax.axis_index("s")
    wid = c * NS + s
    zero16 = jnp.zeros((16,), jnp.float32)

    def zrow(i, carry):
        for jcol in range(DP // 16):
            zbuf[i, pl.ds(jcol * 16, 16)] = zero16
        return carry

    lax.fori_loop(0, ZR, zrow, 0)

    rbase = s * RPT
    for i in range(RPT // ZR):
        pltpu.sync_copy(zbuf, accum.at[pl.ds(rbase + i * ZR, ZR)])

    pltpu.sync_copy(src_hbm.at[wid], srcb)
    pltpu.sync_copy(dst_hbm.at[wid], dstb)
    plsc.subcore_barrier()

    pltpu.async_copy(h_hbm.at[srcb.at[0]], rows[0], gsem[0])

    def pair(i, carry):
        j = i * 2
        pltpu.async_copy(h_hbm.at[srcb.at[j + 1]], rows[1], gsem[1])
        pltpu.make_async_copy(h_hbm.at[srcb.at[j]], rows[0], gsem[0]).wait()
        pltpu.sync_copy(rows[0], accum.at[dstb.at[j]], add=True)

        @pl.when(j + 2 < NCH)
        def _():
            pltpu.async_copy(h_hbm.at[srcb.at[j + 2]], rows[0], gsem[0])

        pltpu.make_async_copy(h_hbm.at[srcb.at[j + 1]], rows[1], gsem[1]).wait()
        pltpu.sync_copy(rows[1], accum.at[dstb.at[j + 1]], add=True)
        return carry

    lax.fori_loop(0, NCH // 2, pair, 0)
    plsc.subcore_barrier()
    pltpu.sync_copy(
        accum.at[pl.ds(rbase, RPT)],
        out_hbm.at[pl.ds(c * NP + rbase, RPT)],
    )


def _tc1_body(x_ref, wsgc_ref, wmlp_ref, degp_ref, h0_ref):
    deg = jnp.sum(degp_ref[...], axis=0) + 1.0
    dinv = lax.rsqrt(deg)
    wf = jnp.dot(wsgc_ref[...], wmlp_ref[...], preferred_element_type=jnp.float32)
    z = jnp.dot(x_ref[...], wf, preferred_element_type=jnp.float32)
    h0_ref[...] = z * dinv[:, None]


def _tc2_body(sa_ref, sb_ref, h0_ref, degp_ref, h1_ref):
    deg = jnp.sum(degp_ref[...], axis=0) + 1.0
    ssum = sa_ref[...] + sb_ref[...] + h0_ref[...]
    h1_ref[...] = ssum / deg[:, None]


def _tc3_body(sa_ref, sb_ref, h1_ref, degp_ref, wmlp_ref, bsgc_ref, bmlp_ref, out_ref):
    deg = jnp.sum(degp_ref[...], axis=0) + 1.0
    dinv = lax.rsqrt(deg)
    r = (sa_ref[...] + sb_ref[...] + h1_ref[...]) * dinv[:, None]
    bf = jnp.dot(bsgc_ref[...], wmlp_ref[...], preferred_element_type=jnp.float32) + bmlp_ref[...]
    out_ref[...] = r[:, :CD] + bf


_row_spec = pl.BlockSpec((BR, DP), lambda i: (i, 0))
_dega_spec = pl.BlockSpec((NW, BR), lambda i: (0, i))
_sa_spec = pl.BlockSpec((BR, DP), lambda i: (i, 0))
_sb_spec = pl.BlockSpec((BR, DP), lambda i: (i + GR, 0))

_tc1 = pl.pallas_call(
    _tc1_body,
    grid=(GR,),
    in_specs=[
        pl.BlockSpec((BR, FD), lambda i: (i, 0)),
        pl.BlockSpec((FD, HD), lambda i: (0, 0)),
        pl.BlockSpec((HD, DP), lambda i: (0, 0)),
        _dega_spec,
    ],
    out_specs=_row_spec,
    out_shape=jax.ShapeDtypeStruct((NN, DP), jnp.float32),
)

_tc2 = pl.pallas_call(
    _tc2_body,
    grid=(GR,),
    in_specs=[_sa_spec, _sb_spec, _row_spec, _dega_spec],
    out_specs=_row_spec,
    out_shape=jax.ShapeDtypeStruct((NN, DP), jnp.float32),
)

_tc3 = pl.pallas_call(
    _tc3_body,
    grid=(GR,),
    in_specs=[
        _sa_spec,
        _sb_spec,
        _row_spec,
        _dega_spec,
        pl.BlockSpec((HD, CD), lambda i: (0, 0)),
        pl.BlockSpec((1, HD), lambda i: (0, 0)),
        pl.BlockSpec((1, CD), lambda i: (0, 0)),
    ],
    out_specs=pl.BlockSpec((BR, CD), lambda i: (i, 0)),
    out_shape=jax.ShapeDtypeStruct((NN, CD), jnp.float32),
)


def kernel(x, edge_index, W_sgc, b_sgc, W_mlp, b_mlp):
    src = edge_index[0]
    dst = edge_index[1]
    wmlp_pad = jnp.pad(W_mlp, ((0, 0), (0, DP - CD)))
    npad = EPAD - NE
    pads = jnp.arange(npad, dtype=jnp.int32)
    srcp = jnp.concatenate([src, pads % NN])
    dstp = jnp.concatenate([dst, NN + pads % (NP - NN)])
    src3 = srcp.reshape(NW, NCH, CHW)
    dst3 = dstp.reshape(NW, NCH, CHW)

    degp = _deg_kernel(dstp)
    h0 = _tc1(x, W_sgc, wmlp_pad, degp)
    s1 = _hop_kernel(h0, src3, dst3)
    h1 = _tc2(s1, s1, h0, degp)
    s2 = _hop_kernel(h1, src3, dst3)
    return _tc3(s2, s2, h1, degp, W_mlp,
                b_sgc.reshape(1, HD), b_mlp.reshape(1, CD))
